# baseline (speedup 1.0000x reference)
import functools

import jax
import jax.numpy as jnp
from jax import lax
from jax.experimental import pallas as pl
from jax.experimental.pallas import tpu as pltpu
from jax.experimental.pallas import tpu_sc as plsc

D = 128
S = 10
SLOTS = S + 1
NC, NS = 2, 16
NW = NC * NS
CH = 16
NB = 3
RPW0 = 1824
RPW1 = 1344
NCH0 = RPW0 // CH
NCH1 = RPW1 // CH
B_PAD = NS * (RPW0 + RPW1)
LANES = 16
KD = D // LANES


def _sc_attention(features, idx_arr):
    mesh = plsc.VectorSubcoreMesh(core_axis_name="c", subcore_axis_name="s")

    @functools.partial(
        pl.kernel,
        mesh=mesh,
        out_type=jax.ShapeDtypeStruct((B_PAD, D), jnp.float32),
        compiler_params=pltpu.CompilerParams(needs_layout_passes=False),
        scratch_types=(
            [pltpu.VMEM((SLOTS, RPW0), jnp.int32)]
            + [pltpu.VMEM((SLOTS, CH, D), jnp.float32)
               for _ in range(NB)]
            + [pltpu.VMEM((CH, D), jnp.float32)
               for _ in range(NB)]
            + [pltpu.SemaphoreType.DMA for _ in range(2 * NB)]
        ),
    )
    def body(feat_hbm, idx_hbm, out_hbm, idx_v, *bufs_and_sems):
        gbufs = bufs_and_sems[:NB]
        obufs = bufs_and_sems[NB:2 * NB]
        gsems = bufs_and_sems[2 * NB:3 * NB]
        osems = bufs_and_sems[3 * NB:]
        cid = lax.axis_index("c")
        sid = lax.axis_index("s")
        wid = cid * NS + sid
        row0 = jnp.where(cid == 0, sid * RPW0, NS * RPW0 + sid * RPW1)
        nch = jnp.where(cid == 0, NCH0, NCH1)
        pltpu.sync_copy(idx_hbm.at[wid], idx_v)
        lanes_iota = lax.broadcasted_iota(jnp.int32, (LANES,), 0)

        def issue_gather(c, b):
            cc = jnp.minimum(c, nch - 1)
            for s in range(SLOTS):
                pltpu.async_copy(feat_hbm.at[idx_v.at[s, pl.ds(cc * CH, CH)]],
                                 gbufs[b].at[s], gsems[b])

        def wait_gather(b):
            for s in range(SLOTS):
                pltpu.make_async_copy(feat_hbm.at[idx_v.at[s, pl.ds(0, CH)]],
                                      gbufs[b].at[s], gsems[b]).wait()

        def issue_out(c, b):
            pltpu.async_copy(
                obufs[b],
                out_hbm.at[pl.ds(row0 + c * CH, CH), :],
                osems[b])

        def wait_out(b):
            pltpu.make_async_copy(
                obufs[b], out_hbm.at[pl.ds(row0, CH), :],
                osems[b]).wait()

        def compute(b):
            gbuf = gbufs[b]
            obuf = obufs[b]

            def row_body(r):
                nf = [gbuf[0, r, pl.ds(k * LANES, LANES)] for k in range(KD)]
                svec = jnp.full((LANES,), -1e30, jnp.float32)
                for s in range(1, SLOTS):
                    acc = nf[0] * gbuf[s, r, pl.ds(0, LANES)]
                    for k in range(1, KD):
                        acc = acc + nf[k] * gbuf[s, r,
                                                 pl.ds(k * LANES, LANES)]
                    tot = jnp.sum(acc)
                    svec = jnp.where(lanes_iota == (s - 1), tot, svec)
                m = jnp.max(svec)
                e = jnp.exp(svec - m)
                z = jnp.sum(e)
                attn = e / jnp.full((LANES,), z)
                av = [
                    attn.at[jnp.full((LANES,), s, jnp.int32)].get(
                        mode="promise_in_bounds")
                    for s in range(S)
                ]
                for k in range(KD):
                    ok = av[0] * gbuf[1, r, pl.ds(k * LANES, LANES)]
                    for s in range(1, S):
                        ok = ok + av[s] * gbuf[s + 1, r,
                                               pl.ds(k * LANES, LANES)]
                    obuf[r, pl.ds(k * LANES, LANES)] = ok

            plsc.parallel_loop(0, CH, unroll=2)(row_body)

        for b in range(NB):
            issue_gather(jnp.int32(b), b)

        for b in range(NB):
            c = jnp.int32(b)
            wait_gather(b)
            compute(b)
            issue_out(c, b)
            issue_gather(c + NB, b)

        def round_body(i, carry):
            for b in range(NB):
                c = NB * i + b
                wait_gather(b)
                wait_out(b)
                compute(b)
                issue_out(c, b)
                issue_gather(c + NB, b)
            return carry

        lax.fori_loop(1, nch // NB, round_body, 0)

        for b in range(NB):
            wait_gather(b)
            wait_out(b)

    return body(features, idx_arr)


def kernel(features, nodes, neigh_idx, num_sample):
    del num_sample
    B = nodes.shape[0]
    nodes_p = jnp.pad(nodes, (0, B_PAD - B))
    neigh_p = jnp.pad(neigh_idx, ((0, B_PAD - B), (0, 0)))
    comb = jnp.concatenate([nodes_p[:, None], neigh_p], axis=1)

    idx0 = (comb[:NS * RPW0].reshape(NS, RPW0, SLOTS)
            .transpose(0, 2, 1))
    idx1 = (comb[NS * RPW0:].reshape(NS, RPW1, SLOTS)
            .transpose(0, 2, 1))
    idx1 = jnp.pad(idx1, ((0, 0), (0, 0), (0, RPW0 - RPW1)))
    idx_arr = jnp.concatenate([idx0, idx1], axis=0)
    out = _sc_attention(features, idx_arr)
    return out[:B]


# speedup vs baseline: 1.8097x; 1.8097x over previous
import functools

import jax
import jax.numpy as jnp
from jax import lax
from jax.experimental import pallas as pl
from jax.experimental.pallas import tpu as pltpu
from jax.experimental.pallas import tpu_sc as plsc

D = 128
S = 10
SLOTS = S + 1
NC, NS = 2, 16
NW = NC * NS
CH = 16
NB = 2
RPW0 = 1792
RPW1 = 1344
NCH0 = RPW0 // CH
NCH1 = RPW1 // CH
B_PAD = NS * (RPW0 + RPW1)
LANES = 16
KD = D // LANES


def _sc_attention(features, idx_arr):
    mesh = plsc.VectorSubcoreMesh(core_axis_name="c", subcore_axis_name="s")

    @functools.partial(
        pl.kernel,
        mesh=mesh,
        out_type=jax.ShapeDtypeStruct((B_PAD, D), jnp.float32),
        compiler_params=pltpu.CompilerParams(needs_layout_passes=False),
        scratch_types=(
            [pltpu.VMEM((SLOTS, RPW0), jnp.int32)]
            + [pltpu.VMEM((SLOTS, CH, D), jnp.float32)
               for _ in range(NB)]
            + [pltpu.VMEM((CH, D), jnp.float32)
               for _ in range(NB)]
            + [pltpu.SemaphoreType.DMA for _ in range(2 * NB)]
        ),
    )
    def body(feat_hbm, idx_hbm, out_hbm, idx_v, *bufs_and_sems):
        gbufs = bufs_and_sems[:NB]
        obufs = bufs_and_sems[NB:2 * NB]
        gsems = bufs_and_sems[2 * NB:3 * NB]
        osems = bufs_and_sems[3 * NB:]
        cid = lax.axis_index("c")
        sid = lax.axis_index("s")
        wid = cid * NS + sid
        row0 = jnp.where(cid == 0, sid * RPW0, NS * RPW0 + sid * RPW1)
        nch = jnp.where(cid == 0, NCH0, NCH1)
        pltpu.sync_copy(idx_hbm.at[wid], idx_v)
        lanes_iota = lax.broadcasted_iota(jnp.int32, (LANES,), 0)

        def issue_gather(c, b):
            cc = jnp.minimum(c, nch - 1)
            for s in range(SLOTS):
                pltpu.async_copy(feat_hbm.at[idx_v.at[s, pl.ds(cc * CH, CH)]],
                                 gbufs[b].at[s], gsems[b])

        def wait_gather(b):
            for s in range(SLOTS):
                pltpu.make_async_copy(feat_hbm.at[idx_v.at[s, pl.ds(0, CH)]],
                                      gbufs[b].at[s], gsems[b]).wait()

        def issue_out(c, b):
            pltpu.async_copy(
                obufs[b],
                out_hbm.at[pl.ds(row0 + c * CH, CH), :],
                osems[b])

        def wait_out(b):
            pltpu.make_async_copy(
                obufs[b], out_hbm.at[pl.ds(row0, CH), :],
                osems[b]).wait()

        def compute(b):
            gbuf = gbufs[b]
            obuf = obufs[b]

            def row_body(r):
                nf = [gbuf[0, r, pl.ds(k * LANES, LANES)] for k in range(KD)]
                svec = jnp.full((LANES,), -1e30, jnp.float32)
                for s in range(1, SLOTS):
                    acc = nf[0] * gbuf[s, r, pl.ds(0, LANES)]
                    for k in range(1, KD):
                        acc = acc + nf[k] * gbuf[s, r,
                                                 pl.ds(k * LANES, LANES)]
                    tot = jnp.sum(acc)
                    svec = jnp.where(lanes_iota == (s - 1), tot, svec)
                m = jnp.max(svec)
                e = jnp.exp(svec - m)
                z = jnp.sum(e)
                attn = e / jnp.full((LANES,), z)
                av = [
                    attn.at[jnp.full((LANES,), s, jnp.int32)].get(
                        mode="promise_in_bounds")
                    for s in range(S)
                ]
                for k in range(KD):
                    ok = av[0] * gbuf[1, r, pl.ds(k * LANES, LANES)]
                    for s in range(1, S):
                        ok = ok + av[s] * gbuf[s + 1, r,
                                               pl.ds(k * LANES, LANES)]
                    obuf[r, pl.ds(k * LANES, LANES)] = ok

            plsc.parallel_loop(0, CH, unroll=2)(row_body)

        for b in range(NB):
            issue_gather(jnp.int32(b), b)

        for b in range(NB):
            c = jnp.int32(b)
            wait_gather(b)
            compute(b)
            issue_out(c, b)
            issue_gather(c + NB, b)

        def round_body(i, carry):
            for b in range(NB):
                c = NB * i + b
                wait_gather(b)
                wait_out(b)
                compute(b)
                issue_out(c, b)
                issue_gather(c + NB, b)
            return carry

        lax.fori_loop(1, nch // NB, round_body, 0)

        for b in range(NB):
            wait_gather(b)
            wait_out(b)

    return body(features, idx_arr)


def kernel(features, nodes, neigh_idx, num_sample):
    del num_sample
    B = nodes.shape[0]
    nodes_p = jnp.pad(nodes, (0, B_PAD - B))
    neigh_p = jnp.pad(neigh_idx, ((0, B_PAD - B), (0, 0)))
    comb = jnp.concatenate([nodes_p[:, None], neigh_p], axis=1)

    idx0 = (comb[:NS * RPW0].reshape(NS, RPW0, SLOTS)
            .transpose(0, 2, 1))
    idx1 = (comb[NS * RPW0:].reshape(NS, RPW1, SLOTS)
            .transpose(0, 2, 1))
    idx1 = jnp.pad(idx1, ((0, 0), (0, 0), (0, RPW0 - RPW1)))
    idx_arr = jnp.concatenate([idx0, idx1], axis=0)
    out = _sc_attention(features, idx_arr)
    return out[:B]
